# baseline (speedup 1.0000x reference)
import functools

import jax
import jax.numpy as jnp
from jax.experimental import pallas as pl
from jax.experimental.pallas import tpu as pltpu

_BOUNDARY_MARGIN = 2.0
_BN = 1024


def _fused_body(x3_ref, xf_ref, cT_ref, W1c_ref, b1c_ref, W2r_ref, b2_ref,
                out_ref, *, n_exp, hid):
    bn = x3_ref.shape[0]
    x3 = x3_ref[...]
    cd2 = jnp.zeros((bn, n_exp), dtype=jnp.float32)
    for i in range(3):
        d = x3[:, i:i + 1] - cT_ref[i:i + 1, :]
        cd2 = cd2 + d * d
    cd = jnp.sqrt(cd2)
    inv = 1.0 / (cd + 1e-8)
    mind = jnp.min(cd, axis=1, keepdims=True)
    inv = jnp.where(cd > _BOUNDARY_MARGIN * mind, 0.0, inv)
    w = inv / jnp.sum(inv, axis=1, keepdims=True)

    h = jnp.dot(xf_ref[...], W1c_ref[...],
                preferred_element_type=jnp.float32) + b1c_ref[...]
    h = jnp.maximum(h, 0.0)
    hw = (h.reshape(bn, n_exp, hid) * w[:, :, None]).reshape(bn, n_exp * hid)
    out = jnp.dot(hw, W2r_ref[...], preferred_element_type=jnp.float32)
    out = out + jnp.dot(w, b2_ref[...], preferred_element_type=jnp.float32)
    out_ref[...] = out


def kernel(x, centroids, W1, b1, W2, b2):
    N = x.shape[0]
    E, D_IN, H = W1.shape
    OUT = W2.shape[-1]
    x3 = x[:, :3]
    xf = x[:, 3:]
    cT = centroids.T
    W1c = jnp.transpose(W1, (1, 0, 2)).reshape(D_IN, E * H)
    b1c = b1.reshape(1, E * H)
    W2r = W2.reshape(E * H, OUT)

    grid = (N // _BN,)
    body = functools.partial(_fused_body, n_exp=E, hid=H)
    return pl.pallas_call(
        body,
        grid=grid,
        in_specs=[
            pl.BlockSpec((_BN, 3), lambda i: (i, 0)),
            pl.BlockSpec((_BN, D_IN), lambda i: (i, 0)),
            pl.BlockSpec((3, E), lambda i: (0, 0)),
            pl.BlockSpec((D_IN, E * H), lambda i: (0, 0)),
            pl.BlockSpec((1, E * H), lambda i: (0, 0)),
            pl.BlockSpec((E * H, OUT), lambda i: (0, 0)),
            pl.BlockSpec((E, OUT), lambda i: (0, 0)),
        ],
        out_specs=pl.BlockSpec((_BN, OUT), lambda i: (i, 0)),
        out_shape=jax.ShapeDtypeStruct((N, OUT), jnp.float32),
        compiler_params=pltpu.CompilerParams(
            dimension_semantics=("parallel",)),
    )(x3, xf, cT, W1c, b1c, W2r, b2)


# speedup vs baseline: 3.2415x; 3.2415x over previous
import functools

import jax
import jax.numpy as jnp
import numpy as np
from jax.experimental import pallas as pl
from jax.experimental.pallas import tpu as pltpu

_BOUNDARY_MARGIN = 2.0
_BN = 1024


def _fused_body(x3T_ref, xfT_ref, cents_ref, W1cT_ref, b1cT_ref, W2bdT_ref,
                RT_ref, S_ref, b2T_ref, outT_ref, *, n_exp):
    bn = x3T_ref.shape[1]
    cd2 = jnp.zeros((n_exp, bn), dtype=jnp.float32)
    for i in range(3):
        d = x3T_ref[i:i + 1, :] - cents_ref[:, i:i + 1]
        cd2 = cd2 + d * d
    cd = jnp.sqrt(cd2)
    inv = 1.0 / (cd + 1e-8)
    mind = jnp.min(cd, axis=0, keepdims=True)
    inv = jnp.where(cd > _BOUNDARY_MARGIN * mind, 0.0, inv)
    wT = inv / jnp.sum(inv, axis=0, keepdims=True)

    hT = jnp.dot(W1cT_ref[...], xfT_ref[...],
                 preferred_element_type=jnp.float32) + b1cT_ref[...]
    hT = jnp.maximum(hT, 0.0).astype(jnp.bfloat16)
    M2T = jnp.dot(W2bdT_ref[...], hT,
                  preferred_element_type=jnp.float32)
    w_expT = jnp.dot(RT_ref[...], wT,
                     preferred_element_type=jnp.float32)
    outT = jnp.dot(S_ref[...], M2T * w_expT,
                   preferred_element_type=jnp.float32)
    outT = outT + jnp.dot(b2T_ref[...], wT,
                          preferred_element_type=jnp.float32)
    outT_ref[...] = outT


def kernel(x, centroids, W1, b1, W2, b2):
    N = x.shape[0]
    E, D_IN, H = W1.shape
    OUT = W2.shape[-1]
    EH, EO = E * H, E * OUT

    x3T = x[:, :3].T
    xfT = x[:, 3:].T.astype(jnp.bfloat16)
    W1cT = (jnp.transpose(W1, (1, 0, 2)).reshape(D_IN, EH)
            .T.astype(jnp.bfloat16))
    b1cT = b1.reshape(EH, 1)
    W2bdT = (jax.vmap(jnp.transpose)(W2)
             .reshape(EO, H))
    W2bdT = (W2bdT[:, None, :] *
             jnp.eye(E, dtype=W2.dtype).repeat(OUT, axis=0)[:, :, None]
             ).reshape(EO, EH).astype(jnp.bfloat16)
    RT = jnp.eye(E, dtype=jnp.float32).repeat(OUT, axis=0)
    S = jnp.tile(jnp.eye(OUT, dtype=jnp.float32), (1, E))
    b2T = b2.T

    grid = (N // _BN,)
    body = functools.partial(_fused_body, n_exp=E)
    outT = pl.pallas_call(
        body,
        grid=grid,
        in_specs=[
            pl.BlockSpec((3, _BN), lambda i: (0, i)),
            pl.BlockSpec((D_IN, _BN), lambda i: (0, i)),
            pl.BlockSpec((E, 3), lambda i: (0, 0)),
            pl.BlockSpec((EH, D_IN), lambda i: (0, 0)),
            pl.BlockSpec((EH, 1), lambda i: (0, 0)),
            pl.BlockSpec((EO, EH), lambda i: (0, 0)),
            pl.BlockSpec((EO, E), lambda i: (0, 0)),
            pl.BlockSpec((OUT, EO), lambda i: (0, 0)),
            pl.BlockSpec((OUT, E), lambda i: (0, 0)),
        ],
        out_specs=pl.BlockSpec((OUT, _BN), lambda i: (0, i)),
        out_shape=jax.ShapeDtypeStruct((OUT, N), jnp.float32),
        compiler_params=pltpu.CompilerParams(
            dimension_semantics=("parallel",)),
    )(x3T, xfT, centroids, W1cT, b1cT, W2bdT, RT, S, b2T)
    return outT.T
